# baseline (speedup 1.0000x reference)
import functools

import jax
import jax.numpy as jnp
from jax import lax
from jax.experimental import pallas as pl
from jax.experimental.pallas import tpu as pltpu
from jax.experimental.pallas import tpu_sc as plsc

_VOCAB = 1000000
_D = 32
_BATCH = 16384
_HIST = 50
_NC, _NS = 2, 16
_NW = _NC * _NS
_BW = _BATCH // _NW
_CHUNK = 128
_KPH = _BW // _CHUNK
_NCH = _HIST * _KPH
_NBUF = 12
_L = 16

_mesh = plsc.VectorSubcoreMesh(core_axis_name="c", subcore_axis_name="s")


@functools.partial(
    pl.kernel,
    out_type=jax.ShapeDtypeStruct((_HIST, _D, _BATCH), jnp.float32),
    mesh=_mesh,
    scratch_types=[
        pltpu.VMEM((_HIST, _BW), jnp.int32),
        pltpu.VMEM((_NBUF * _CHUNK, _D), jnp.float32),
        pltpu.VMEM((_D, _CHUNK), jnp.float32),
        pltpu.VMEM((_D, _CHUNK), jnp.float32),
        pltpu.SemaphoreType.DMA,
        pltpu.SemaphoreType.DMA,
    ],
    compiler_params=pltpu.CompilerParams(
        use_tc_tiling_on_sc=False, needs_layout_passes=False),
)
def _sc_gather(idx_hbm, table_hbm, out_hbm, idx_v, gbuf, tch0, tch1,
               gsem, ssem):
    wid = lax.axis_index("s") * _NC + lax.axis_index("c")
    b0 = wid * _BW
    pltpu.sync_copy(idx_hbm.at[:, pl.ds(b0, _BW)], idx_v)

    def gsrc(j):
        h = lax.div(j, _KPH)
        k = lax.rem(j, _KPH)
        return table_hbm.at[idx_v.at[h, pl.ds(k * _CHUNK, _CHUNK)]]

    def gdst(slot):
        return gbuf.at[pl.ds(slot * _CHUNK, _CHUNK)]

    for j in range(_NBUF - 1):
        pltpu.async_copy(gsrc(j), gdst(j), gsem)

    iota = lax.iota(jnp.int32, 16)
    cols = [jnp.full((16,), d, jnp.int32) for d in range(_D)]

    def body(g, carry):
        for u, tch in ((0, tch0), (1, tch1)):
            j = 2 * g + u
            slot = lax.rem(j, _NBUF)
            h = lax.div(j, _KPH)
            k = lax.rem(j, _KPH)
            pltpu.make_async_copy(gsrc(j), gdst(slot), gsem).wait()

            @pl.when(g >= 1)
            def _drain_store():
                pltpu.make_async_copy(
                    tch0, out_hbm.at[0, :, pl.ds(b0, _CHUNK)], ssem).wait()

            base = iota + slot * _CHUNK
            for k16 in range(_CHUNK // _L):
                rows = base + k16 * _L
                for d in range(_D):
                    v = plsc.load_gather(gbuf, [rows, cols[d]])
                    tch[d, pl.ds(k16 * _L, _L)] = v

            @pl.when(j + _NBUF - 1 < _NCH)
            def _start_next():
                pltpu.async_copy(
                    gsrc(j + _NBUF - 1),
                    gdst(lax.rem(j + _NBUF - 1, _NBUF)), gsem)

            pltpu.async_copy(
                tch, out_hbm.at[h, :, pl.ds(b0 + k * _CHUNK, _CHUNK)], ssem)

        return carry

    lax.fori_loop(0, _NCH // 2, body, 0)

    for _ in range(2):
        pltpu.make_async_copy(
            tch0, out_hbm.at[0, :, pl.ds(b0, _CHUNK)], ssem).wait()


def kernel(input_ids, table):
    idx_t = input_ids.astype(jnp.int32).T
    out = _sc_gather(idx_t, table)
    return out.transpose(2, 0, 1)


# speedup vs baseline: 1.2665x; 1.2665x over previous
import functools

import jax
import jax.numpy as jnp
from jax import lax
from jax.experimental import pallas as pl
from jax.experimental.pallas import tpu as pltpu
from jax.experimental.pallas import tpu_sc as plsc

_VOCAB = 1000000
_D = 32
_BATCH = 16384
_HIST = 50
_NC, _NS = 2, 16
_NW = _NC * _NS
_BW = _BATCH // _NW
_CHUNK = 128
_KPH = _BW // _CHUNK
_NCH = _HIST * _KPH
_NBUF = 12
_L = 16

_mesh = plsc.VectorSubcoreMesh(core_axis_name="c", subcore_axis_name="s")


@functools.partial(
    pl.kernel,
    out_type=jax.ShapeDtypeStruct((_HIST, _D, _BATCH), jnp.float32),
    mesh=_mesh,
    scratch_types=[
        pltpu.VMEM((_HIST, _BW), jnp.int32),
        pltpu.VMEM((_NBUF * _CHUNK, _D), jnp.float32),
        pltpu.VMEM((_D, _CHUNK), jnp.float32),
        pltpu.VMEM((_D, _CHUNK), jnp.float32),
        pltpu.SemaphoreType.DMA,
        pltpu.SemaphoreType.DMA,
    ],
    compiler_params=pltpu.CompilerParams(
        use_tc_tiling_on_sc=False, needs_layout_passes=False),
)
def _sc_gather(idx_hbm, table_hbm, out_hbm, idx_v, gbuf, tch0, tch1,
               gsem, ssem):
    wid = lax.axis_index("s") * _NC + lax.axis_index("c")
    b0 = wid * _BW
    pltpu.sync_copy(idx_hbm.at[:, pl.ds(b0, _BW)], idx_v)

    def gsrc(j):
        h = lax.div(j, _KPH)
        k = lax.rem(j, _KPH)
        return table_hbm.at[idx_v.at[h, pl.ds(k * _CHUNK, _CHUNK)]]

    def gdst(slot):
        return gbuf.at[pl.ds(slot * _CHUNK, _CHUNK)]

    for j in range(_NBUF - 1):
        pltpu.async_copy(gsrc(j), gdst(j), gsem)

    iota = lax.iota(jnp.int32, 16)
    cols = [jnp.full((16,), d, jnp.int32) for d in range(_D)]

    def body(g, carry):
        for u, tch in ((0, tch0), (1, tch1)):
            j = 2 * g + u
            slot = lax.rem(j, _NBUF)
            h = lax.div(j, _KPH)
            k = lax.rem(j, _KPH)
            pltpu.make_async_copy(gsrc(j), gdst(slot), gsem).wait()

            @pl.when(g >= 1)
            def _drain_store():
                pltpu.make_async_copy(
                    tch0, out_hbm.at[0, :, pl.ds(b0, _CHUNK)], ssem).wait()

            base = iota + slot * _CHUNK
            for k16 in range(_CHUNK // _L):
                rows = base + k16 * _L
                for d0 in range(0, _D, 8):
                    vs = [plsc.load_gather(gbuf, [rows, cols[d0 + i]])
                          for i in range(8)]
                    for i in range(8):
                        tch[d0 + i, pl.ds(k16 * _L, _L)] = vs[i]

            @pl.when(j + _NBUF - 1 < _NCH)
            def _start_next():
                pltpu.async_copy(
                    gsrc(j + _NBUF - 1),
                    gdst(lax.rem(j + _NBUF - 1, _NBUF)), gsem)

            pltpu.async_copy(
                tch, out_hbm.at[h, :, pl.ds(b0 + k * _CHUNK, _CHUNK)], ssem)

        return carry

    lax.fori_loop(0, _NCH // 2, body, 0)

    for _ in range(2):
        pltpu.make_async_copy(
            tch0, out_hbm.at[0, :, pl.ds(b0, _CHUNK)], ssem).wait()


def kernel(input_ids, table):
    idx_t = input_ids.astype(jnp.int32).T
    out = _sc_gather(idx_t, table)
    return out.transpose(2, 0, 1)
